# baseline (speedup 1.0000x reference)
import jax
import jax.numpy as jnp
import numpy as np
from jax.experimental import pallas as pl
from jax.experimental.pallas import tpu as pltpu

_B, _T, _P, _DS, _DO = 16, 32, 1024, 32, 32
_H = 4 * _DS
_RC = 32
_TINY = np.float32(np.finfo(np.float32).tiny)


def _tf_bits(k0, k1, m):
    ks2 = k0 ^ k1 ^ jnp.uint32(0x1BD11BDA)
    ks = (k0, k1, ks2)
    rot = ((13, 15, 26, 6), (17, 29, 16, 24))
    x0 = jnp.zeros_like(m) + k0
    x1 = m + k1
    for r in range(5):
        for d in rot[r % 2]:
            x0 = x0 + x1
            x1 = (x1 << d) | (x1 >> (32 - d))
            x1 = x1 ^ x0
        x0 = x0 + ks[(r + 1) % 3]
        x1 = x1 + ks[(r + 2) % 3] + jnp.uint32(r + 1)
    return x0 ^ x1


def _sig(x):
    return jax.nn.sigmoid(x)


def _pf_kernel(keys_ref, obs_ref, noise_ref, K1n_ref, K1o_ref, R1_ref, b1_ref,
               K2_ref, R2_ref, b2_ref, Wmo_ref, WmpT_ref, bm_ref,
               p_out_ref, w_out_ref, S_ref, Sg_ref, wlog_ref):
    t = pl.program_id(0)
    k0 = keys_ref[t, 0]
    k1 = keys_ref[t, 1]

    def bbody(b, carry):
        bu = b.astype(jnp.uint32)

        @pl.when(t == 0)
        def _():
            Sg_ref[...] = jnp.zeros((_P, _H), jnp.float32)

        @pl.when(t > 0)
        def _():
            lgrow = wlog_ref[pl.ds(b, 1), :]
            nsrow = -jnp.exp(-lgrow)

            def cbody(c, carry2):
                cu = c.astype(jnp.uint32)
                row = jax.lax.broadcasted_iota(jnp.uint32, (_RC, _P), 0)
                colu = jax.lax.broadcasted_iota(jnp.uint32, (_RC, _P), 1)
                m = ((bu << 20) + ((cu * jnp.uint32(_RC) + row) << 10) + colu)
                bits = _tf_bits(k0, k1, m)
                fb = jax.lax.bitcast_convert_type(
                    (bits >> 9) | jnp.uint32(0x3F800000), jnp.float32)
                fb = fb - jnp.float32(1.0)
                u = jnp.maximum(fb, _TINY)
                v = jnp.log(u) * nsrow
                mn = jnp.min(v, axis=1, keepdims=True)
                coli = jax.lax.broadcasted_iota(jnp.int32, (_RC, _P), 1)
                idx = jnp.min(jnp.where(v == mn, coli, jnp.int32(_P)),
                              axis=1, keepdims=True)
                oh = (coli == idx).astype(jnp.float32)
                Sb = S_ref[pl.ds(b, 1)].reshape(_P, _H)
                Sg_ref[pl.ds(c * _RC, _RC), :] = jnp.dot(
                    oh, Sb, preferred_element_type=jnp.float32)
                return carry2

            jax.lax.fori_loop(0, _P // _RC, cbody, 0)

        Sg = Sg_ref[...]
        h1g = Sg[:, 0:_DS]
        c1g = Sg[:, _DS:2 * _DS]
        h2g = Sg[:, 2 * _DS:3 * _DS]
        c2g = Sg[:, 3 * _DS:4 * _DS]
        nz = noise_ref[0, pl.ds(b * _P, _P), :]
        obs_row = obs_ref[0, pl.ds(b, 1), :]
        xo1 = jnp.dot(obs_row, K1o_ref[...],
                      preferred_element_type=jnp.float32) + b1_ref[...]
        z1 = (jnp.dot(nz, K1n_ref[...], preferred_element_type=jnp.float32)
              + jnp.dot(h1g, R1_ref[...], preferred_element_type=jnp.float32)
              + xo1)
        c1n = _sig(z1[:, _DS:2 * _DS]) * c1g + \
            _sig(z1[:, 0:_DS]) * jnp.tanh(z1[:, 2 * _DS:3 * _DS])
        h1n = _sig(z1[:, 3 * _DS:4 * _DS]) * jnp.tanh(c1n)
        z2 = (jnp.dot(h1n, K2_ref[...], preferred_element_type=jnp.float32)
              + jnp.dot(h2g, R2_ref[...], preferred_element_type=jnp.float32)
              + b2_ref[...])
        c2n = _sig(z2[:, _DS:2 * _DS]) * c2g + \
            _sig(z2[:, 0:_DS]) * jnp.tanh(z2[:, 2 * _DS:3 * _DS])
        h2n = _sig(z2[:, 3 * _DS:4 * _DS]) * jnp.tanh(c2n)
        wofs = jnp.dot(obs_row, Wmo_ref[...],
                       preferred_element_type=jnp.float32) + bm_ref[...]
        wrow = jax.lax.dot_general(
            WmpT_ref[...], h2n, (((1,), (1,)), ((), ())),
            preferred_element_type=jnp.float32)
        wlog_ref[pl.ds(b, 1), :] = wrow + wofs
        S_ref[pl.ds(b, 1)] = jnp.concatenate(
            [h1n, c1n, h2n, c2n], axis=1).reshape(1, _P, _H)

        @pl.when(t == _T - 1)
        def _():
            p_out_ref[pl.ds(b, 1)] = h2n.reshape(1, _P, _DS)
            w_out_ref[pl.ds(b, 1), :] = wrow + wofs

        return carry

    jax.lax.fori_loop(0, _B, bbody, 0)


def kernel(observations, K1, R1, b1, K2, R2, b2, Wm, bm):
    key = jax.random.key(1)
    step_keys = jax.random.split(key, _T)
    both = jax.vmap(jax.random.split)(step_keys)
    keys_u32 = jax.random.key_data(both[:, 0]).astype(jnp.uint32)
    noise = jax.vmap(
        lambda k: jax.random.normal(k, (_B, _P, _DS), jnp.float32)
    )(both[:, 1]).reshape(_T, _B * _P, _DS)
    obs_t = jnp.transpose(observations, (1, 0, 2))
    K1n, K1o = K1[:_DS], K1[_DS:]
    b1r = b1.reshape(1, _H)
    b2r = b2.reshape(1, _H)
    Wmo = Wm[:_DO]
    WmpT = Wm[_DO:].T
    bmr = bm.reshape(1, 1)

    fixed = lambda *shape: pl.BlockSpec(shape, lambda t, *_: (0,) * len(shape))
    grid_spec = pltpu.PrefetchScalarGridSpec(
        num_scalar_prefetch=1,
        grid=(_T,),
        in_specs=[
            pl.BlockSpec((1, _B, _DO), lambda t, *_: (t, 0, 0)),
            pl.BlockSpec((1, _B * _P, _DS), lambda t, *_: (t, 0, 0)),
            fixed(_DS, _H),
            fixed(_DO, _H),
            fixed(_DS, _H),
            fixed(1, _H),
            fixed(_DS, _H),
            fixed(_DS, _H),
            fixed(1, _H),
            fixed(_DO, 1),
            fixed(1, _DS),
            fixed(1, 1),
        ],
        out_specs=[
            pl.BlockSpec((_B, _P, _DS), lambda t, *_: (0, 0, 0)),
            pl.BlockSpec((_B, _P), lambda t, *_: (0, 0)),
        ],
        scratch_shapes=[
            pltpu.VMEM((_B, _P, _H), jnp.float32),
            pltpu.VMEM((_P, _H), jnp.float32),
            pltpu.VMEM((_B, _P), jnp.float32),
        ],
    )
    p_out, w_out = pl.pallas_call(
        _pf_kernel,
        grid_spec=grid_spec,
        out_shape=[
            jax.ShapeDtypeStruct((_B, _P, _DS), jnp.float32),
            jax.ShapeDtypeStruct((_B, _P), jnp.float32),
        ],
        compiler_params=pltpu.CompilerParams(
            dimension_semantics=("arbitrary",)),
    )(keys_u32, obs_t, noise, K1n, K1o, R1, b1r, K2, R2, b2r, Wmo, WmpT, bmr)
    return p_out, w_out.reshape(_B, _P, 1)


# speedup vs baseline: 1.3478x; 1.3478x over previous
import jax
import jax.numpy as jnp
import numpy as np
from jax.experimental import pallas as pl
from jax.experimental.pallas import tpu as pltpu

_B, _T, _P, _DS, _DO = 16, 32, 1024, 32, 32
_H = 4 * _DS
_RC = 128
_TINY = np.float32(np.finfo(np.float32).tiny)


def _tf_bits(k0, k1, m):
    ks2 = k0 ^ k1 ^ jnp.uint32(0x1BD11BDA)
    ks = (k0, k1, ks2)
    rot = ((13, 15, 26, 6), (17, 29, 16, 24))
    x0 = jnp.zeros_like(m) + k0
    x1 = m + k1
    for r in range(5):
        for d in rot[r % 2]:
            x0 = x0 + x1
            x1 = (x1 << d) | (x1 >> (32 - d))
            x1 = x1 ^ x0
        x0 = x0 + ks[(r + 1) % 3]
        x1 = x1 + ks[(r + 2) % 3] + jnp.uint32(r + 1)
    return x0 ^ x1


def _sig(x):
    return jax.nn.sigmoid(x)


def _pf_kernel(keys_ref, obs_ref, noise_ref, K1n_ref, K1o_ref, R1_ref, b1_ref,
               K2_ref, R2_ref, b2_ref, Wmo_ref, WmpT_ref, bm_ref,
               p_out_ref, w_out_ref, S_ref, Sg_ref, wlog_ref):
    t = pl.program_id(0)
    k0 = keys_ref[t, 0]
    k1 = keys_ref[t, 1]

    def bbody(b, carry):
        bu = b.astype(jnp.uint32)

        @pl.when(t == 0)
        def _():
            Sg_ref[...] = jnp.zeros((_P, _H), jnp.float32)

        @pl.when(t > 0)
        def _():
            lgrow = wlog_ref[pl.ds(b, 1), :]
            nsrow = -jnp.exp(-lgrow)

            def cbody(c, carry2):
                cu = c.astype(jnp.uint32)
                row = jax.lax.broadcasted_iota(jnp.uint32, (_RC, _P), 0)
                colu = jax.lax.broadcasted_iota(jnp.uint32, (_RC, _P), 1)
                m = ((bu << 20) + ((cu * jnp.uint32(_RC) + row) << 10) + colu)
                bits = _tf_bits(k0, k1, m)
                fb = jax.lax.bitcast_convert_type(
                    (bits >> 9) | jnp.uint32(0x3F800000), jnp.float32)
                fb = fb - jnp.float32(1.0)
                u = jnp.maximum(fb, _TINY)
                v = jnp.log(u) * nsrow
                mn = jnp.min(v, axis=1, keepdims=True)
                coli = jax.lax.broadcasted_iota(jnp.int32, (_RC, _P), 1)
                idx = jnp.min(jnp.where(v == mn, coli, jnp.int32(_P)),
                              axis=1, keepdims=True)
                oh = (coli == idx).astype(jnp.float32)
                Sb = S_ref[pl.ds(b, 1)].reshape(_P, _H)
                Sg_ref[pl.ds(c * _RC, _RC), :] = jnp.dot(
                    oh, Sb, preferred_element_type=jnp.float32)
                return carry2

            jax.lax.fori_loop(0, _P // _RC, cbody, 0)

        Sg = Sg_ref[...]
        h1g = Sg[:, 0:_DS]
        c1g = Sg[:, _DS:2 * _DS]
        h2g = Sg[:, 2 * _DS:3 * _DS]
        c2g = Sg[:, 3 * _DS:4 * _DS]
        nz = noise_ref[0, pl.ds(b * _P, _P), :]
        obs_row = obs_ref[0, pl.ds(b, 1), :]
        xo1 = jnp.dot(obs_row, K1o_ref[...],
                      preferred_element_type=jnp.float32) + b1_ref[...]
        z1 = (jnp.dot(nz, K1n_ref[...], preferred_element_type=jnp.float32)
              + jnp.dot(h1g, R1_ref[...], preferred_element_type=jnp.float32)
              + xo1)
        c1n = _sig(z1[:, _DS:2 * _DS]) * c1g + \
            _sig(z1[:, 0:_DS]) * jnp.tanh(z1[:, 2 * _DS:3 * _DS])
        h1n = _sig(z1[:, 3 * _DS:4 * _DS]) * jnp.tanh(c1n)
        z2 = (jnp.dot(h1n, K2_ref[...], preferred_element_type=jnp.float32)
              + jnp.dot(h2g, R2_ref[...], preferred_element_type=jnp.float32)
              + b2_ref[...])
        c2n = _sig(z2[:, _DS:2 * _DS]) * c2g + \
            _sig(z2[:, 0:_DS]) * jnp.tanh(z2[:, 2 * _DS:3 * _DS])
        h2n = _sig(z2[:, 3 * _DS:4 * _DS]) * jnp.tanh(c2n)
        wofs = jnp.dot(obs_row, Wmo_ref[...],
                       preferred_element_type=jnp.float32) + bm_ref[...]
        wrow = jax.lax.dot_general(
            WmpT_ref[...], h2n, (((1,), (1,)), ((), ())),
            preferred_element_type=jnp.float32)
        wlog_ref[pl.ds(b, 1), :] = wrow + wofs
        S_ref[pl.ds(b, 1)] = jnp.concatenate(
            [h1n, c1n, h2n, c2n], axis=1).reshape(1, _P, _H)

        @pl.when(t == _T - 1)
        def _():
            p_out_ref[pl.ds(b, 1)] = h2n.reshape(1, _P, _DS)
            w_out_ref[pl.ds(b, 1), :] = wrow + wofs

        return carry

    jax.lax.fori_loop(0, _B, bbody, 0)


def kernel(observations, K1, R1, b1, K2, R2, b2, Wm, bm):
    key = jax.random.key(1)
    step_keys = jax.random.split(key, _T)
    both = jax.vmap(jax.random.split)(step_keys)
    keys_u32 = jax.random.key_data(both[:, 0]).astype(jnp.uint32)
    noise = jax.vmap(
        lambda k: jax.random.normal(k, (_B, _P, _DS), jnp.float32)
    )(both[:, 1]).reshape(_T, _B * _P, _DS)
    obs_t = jnp.transpose(observations, (1, 0, 2))
    K1n, K1o = K1[:_DS], K1[_DS:]
    b1r = b1.reshape(1, _H)
    b2r = b2.reshape(1, _H)
    Wmo = Wm[:_DO]
    WmpT = Wm[_DO:].T
    bmr = bm.reshape(1, 1)

    fixed = lambda *shape: pl.BlockSpec(shape, lambda t, *_: (0,) * len(shape))
    grid_spec = pltpu.PrefetchScalarGridSpec(
        num_scalar_prefetch=1,
        grid=(_T,),
        in_specs=[
            pl.BlockSpec((1, _B, _DO), lambda t, *_: (t, 0, 0)),
            pl.BlockSpec((1, _B * _P, _DS), lambda t, *_: (t, 0, 0)),
            fixed(_DS, _H),
            fixed(_DO, _H),
            fixed(_DS, _H),
            fixed(1, _H),
            fixed(_DS, _H),
            fixed(_DS, _H),
            fixed(1, _H),
            fixed(_DO, 1),
            fixed(1, _DS),
            fixed(1, 1),
        ],
        out_specs=[
            pl.BlockSpec((_B, _P, _DS), lambda t, *_: (0, 0, 0)),
            pl.BlockSpec((_B, _P), lambda t, *_: (0, 0)),
        ],
        scratch_shapes=[
            pltpu.VMEM((_B, _P, _H), jnp.float32),
            pltpu.VMEM((_P, _H), jnp.float32),
            pltpu.VMEM((_B, _P), jnp.float32),
        ],
    )
    p_out, w_out = pl.pallas_call(
        _pf_kernel,
        grid_spec=grid_spec,
        out_shape=[
            jax.ShapeDtypeStruct((_B, _P, _DS), jnp.float32),
            jax.ShapeDtypeStruct((_B, _P), jnp.float32),
        ],
        compiler_params=pltpu.CompilerParams(
            dimension_semantics=("arbitrary",)),
    )(keys_u32, obs_t, noise, K1n, K1o, R1, b1r, K2, R2, b2r, Wmo, WmpT, bmr)
    return p_out, w_out.reshape(_B, _P, 1)
